# baseline (speedup 1.0000x reference)
import functools

import jax
import jax.numpy as jnp
from jax import lax
from jax.experimental import pallas as pl
from jax.experimental.pallas import tpu as pltpu
from jax.experimental.pallas import tpu_sc as plsc

B = 16384
F = 64
H = 32
NC = 2
NS = 16
NW = NC * NS
BPW = B // NW
CHUNK = 128
NCH = BPW // CHUNK


def _sc_gather_body(gu_t, gi_t, mu_t, mi_t, uidx, iidx,
                    gu_o, gi_o, mu_o, mi_o,
                    uv, iv, gu_v, gi_v, mu_v, mi_v, sem):
  wid = lax.axis_index("s") * NC + lax.axis_index("c")
  base = wid * BPW
  pltpu.sync_copy(uidx.at[wid], uv)
  pltpu.sync_copy(iidx.at[wid], iv)
  copies = []
  for j in range(NCH):
    sl = pl.ds(j * CHUNK, CHUNK)
    copies.append(pltpu.async_copy(gu_t.at[uv.at[j]], gu_v.at[sl], sem))
    copies.append(pltpu.async_copy(gi_t.at[iv.at[j]], gi_v.at[sl], sem))
    copies.append(pltpu.async_copy(mu_t.at[uv.at[j]], mu_v.at[sl], sem))
    copies.append(pltpu.async_copy(mi_t.at[iv.at[j]], mi_v.at[sl], sem))
  for c in copies:
    c.wait()
  pltpu.sync_copy(gu_v, gu_o.at[pl.ds(base, BPW)])
  pltpu.sync_copy(gi_v, gi_o.at[pl.ds(base, BPW)])
  pltpu.sync_copy(mu_v, mu_o.at[pl.ds(base, BPW)])
  pltpu.sync_copy(mi_v, mi_o.at[pl.ds(base, BPW)])


@functools.cache
def _sc_gather():
  return pl.kernel(
      _sc_gather_body,
      out_type=(
          jax.ShapeDtypeStruct((B, F), jnp.float32),
          jax.ShapeDtypeStruct((B, F), jnp.float32),
          jax.ShapeDtypeStruct((B, H), jnp.float32),
          jax.ShapeDtypeStruct((B, H), jnp.float32),
      ),
      mesh=plsc.VectorSubcoreMesh(core_axis_name="c", subcore_axis_name="s"),
      compiler_params=pltpu.CompilerParams(use_tc_tiling_on_sc=False),
      scratch_types=[
          pltpu.VMEM((NCH, CHUNK), jnp.int32),
          pltpu.VMEM((NCH, CHUNK), jnp.int32),
          pltpu.VMEM((BPW, F), jnp.float32),
          pltpu.VMEM((BPW, F), jnp.float32),
          pltpu.VMEM((BPW, H), jnp.float32),
          pltpu.VMEM((BPW, H), jnp.float32),
          pltpu.SemaphoreType.DMA,
      ],
  )


def _sigmoid(x):
  return 1.0 / (1.0 + jnp.exp(-x))


BLK = 2048


def _tc_dense_body(gu, gi, mu, mi, w1a, w1b, w2, w3, w4, b1, b2, b3, b4,
                   ow, ob, out):
  gmf = _sigmoid(jnp.sum(gu[...] * gi[...], axis=1, keepdims=True))
  v = jnp.maximum(
      jnp.dot(mu[...], w1a[...], preferred_element_type=jnp.float32)
      + jnp.dot(mi[...], w1b[...], preferred_element_type=jnp.float32)
      + b1[...], 0.0)
  v = jnp.maximum(
      jnp.dot(v, w2[...], preferred_element_type=jnp.float32) + b2[...], 0.0)
  v = jnp.maximum(
      jnp.dot(v, w3[...], preferred_element_type=jnp.float32) + b3[...], 0.0)
  mlp = _sigmoid(jnp.sum(v * w4[...], axis=1, keepdims=True) + b4[...])
  oww = ow[...]
  out[...] = _sigmoid(gmf * oww[0:1, 0:1] + mlp * oww[0:1, 1:2] + ob[...])


def _tc_dense(gu, gi, mu, mi, w1a, w1b, w2, w3, w4, b1, b2, b3, b4, ow, ob):
  full = lambda shape: pl.BlockSpec(shape, lambda i: (0, 0))
  return pl.pallas_call(
      _tc_dense_body,
      grid=(B // BLK,),
      in_specs=[
          pl.BlockSpec((BLK, F), lambda i: (i, 0)),
          pl.BlockSpec((BLK, F), lambda i: (i, 0)),
          pl.BlockSpec((BLK, H), lambda i: (i, 0)),
          pl.BlockSpec((BLK, H), lambda i: (i, 0)),
          full((H, F)),
          full((H, F)),
          full((F, F)),
          full((F, F)),
          full((1, F)),
          full((1, F)),
          full((1, F)),
          full((1, F)),
          full((1, 1)),
          full((1, 2)),
          full((1, 1)),
      ],
      out_specs=pl.BlockSpec((BLK, 1), lambda i: (i, 0)),
      out_shape=jax.ShapeDtypeStruct((B, 1), jnp.float32),
  )(gu, gi, mu, mi, w1a, w1b, w2, w3, w4, b1, b2, b3, b4, ow, ob)


@jax.jit
def kernel(user_ids, item_ids, gmf_user_emb, gmf_item_emb, mlp_user_emb,
           mlp_item_emb, fc_w1, fc_b1, fc_w2, fc_b2, fc_w3, fc_b3,
           mlp_out_w, mlp_out_b, out_w, out_b):
  uidx = jnp.asarray(user_ids, jnp.int32).reshape(NW, NCH, CHUNK)
  iidx = jnp.asarray(item_ids, jnp.int32).reshape(NW, NCH, CHUNK)
  gu, gi, mu, mi = _sc_gather()(gmf_user_emb, gmf_item_emb, mlp_user_emb,
                                mlp_item_emb, uidx, iidx)
  w1a = fc_w1[:, :H].T
  w1b = fc_w1[:, H:].T
  w2 = fc_w2.T
  w3 = fc_w3.T
  w4 = mlp_out_w.reshape(1, F)
  b1 = fc_b1.reshape(1, F)
  b2 = fc_b2.reshape(1, F)
  b3 = fc_b3.reshape(1, F)
  b4 = mlp_out_b.reshape(1, 1)
  ob = out_b.reshape(1, 1)
  return _tc_dense(gu, gi, mu, mi, w1a, w1b, w2, w3, w4, b1, b2, b3, b4,
                   out_w, ob)
